# Initial kernel scaffold; baseline (speedup 1.0000x reference)
import jax
import jax.numpy as jnp
from jax.experimental import pallas as pl


def kernel(local_x, local_edge_index, node_cluster, node_ratio, voxel_x, voxel_edge_index, voxel_level, cross_edge_index, program_noise, voxel_noise, params):
    raise NotImplementedError("write your pallas kernel here")


# speedup vs baseline: 2.1578x; 2.1578x over previous
import functools

import jax
import jax.numpy as jnp
from jax import lax
from jax.experimental import pallas as pl
from jax.experimental.pallas import tpu as pltpu
from jax.experimental.pallas import tpu_sc as plsc

H = 128
N_P = 10000
N_V = 10000
N_C = 500
P_STEPS = 3
V_STEPS = 4
E_P = 160000
E_V = 320000
E_C = 320000

NCORES = 2
NSUB = 16
NW = NCORES * NSUB
CHUNK = 128
NACC = 10240

EP_PAD = 163840
EV_PAD = 327680
EC_PAD = 327680

BLK = 400
NBLK = 25

f32 = jnp.float32
i32 = jnp.int32


def _sc_mesh():
    return plsc.VectorSubcoreMesh(core_axis_name="c", subcore_axis_name="s")


def _fill_rows(buf, nrows, width, val):
    def row(e, carry):
        for j in range(width // 16):
            buf[e, pl.ds(j * 16, 16)] = jnp.full((16,), val, f32)
        return carry
    lax.fori_loop(0, nrows, row, 0)


@functools.cache
def _edge_msg_call(epad):
    epw = epad // NW
    nchunks = epw // CHUNK
    rpt = NACC // NSUB

    @functools.partial(
        pl.kernel,
        out_type=jax.ShapeDtypeStruct((NCORES, NACC, H), f32),
        mesh=_sc_mesh(),
        scratch_types=[
            pltpu.VMEM((CHUNK,), i32),
            pltpu.VMEM((CHUNK,), i32),
            pltpu.VMEM((CHUNK, H), f32),
            pltpu.VMEM((CHUNK, H), f32),
            pltpu.VMEM_SHARED((NACC, H), f32),
            pltpu.SemaphoreType.DMA,
            pltpu.SemaphoreType.DMA,
        ],
    )
    def edge_msg(a_hbm, b_hbm, dst_hbm, src_hbm, out_hbm,
                 idx_d, idx_s, ag, bg, acc, sem_a, sem_b):
        cid = lax.axis_index("c")
        sid = lax.axis_index("s")
        wid = sid * NCORES + cid

        _fill_rows(ag, CHUNK, H, 0.0)
        for r in range(rpt // CHUNK):
            pltpu.sync_copy(ag, acc.at[pl.ds(sid * rpt + r * CHUNK, CHUNK)])
        plsc.subcore_barrier()

        base0 = wid * epw

        def chunk_body(ci, carry):
            base = base0 + ci * CHUNK
            pltpu.sync_copy(dst_hbm.at[pl.ds(base, CHUNK)], idx_d)
            pltpu.sync_copy(src_hbm.at[pl.ds(base, CHUNK)], idx_s)
            ca = pltpu.async_copy(a_hbm.at[idx_d], ag, sem_a)
            cb = pltpu.async_copy(b_hbm.at[idx_s], bg, sem_b)
            ca.wait()
            cb.wait()

            def edge(e, c2):
                for j in range(H // 16):
                    z = ag[e, pl.ds(j * 16, 16)] + bg[e, pl.ds(j * 16, 16)]
                    ag[e, pl.ds(j * 16, 16)] = jnp.maximum(z, 0.01 * z)
                return c2
            lax.fori_loop(0, CHUNK, edge, 0)
            pltpu.sync_copy(ag, acc.at[idx_d], add=True)
            return carry
        lax.fori_loop(0, nchunks, chunk_body, 0)

        plsc.subcore_barrier()
        for r in range(rpt // CHUNK):
            row0 = sid * rpt + r * CHUNK
            pltpu.sync_copy(acc.at[pl.ds(row0, CHUNK)],
                            out_hbm.at[cid, pl.ds(row0, CHUNK)])

    return edge_msg


@functools.cache
def _count_call(epad):
    epw = epad // NW
    nchunks = epw // CHUNK
    rpt = NACC // NSUB

    @functools.partial(
        pl.kernel,
        out_type=jax.ShapeDtypeStruct((NCORES, NACC, 16), f32),
        mesh=_sc_mesh(),
        scratch_types=[
            pltpu.VMEM((CHUNK,), i32),
            pltpu.VMEM((CHUNK, 16), f32),
            pltpu.VMEM_SHARED((NACC, 16), f32),
        ],
    )
    def count(idx_hbm, out_hbm, idx_v, ones_v, acc):
        cid = lax.axis_index("c")
        sid = lax.axis_index("s")
        wid = sid * NCORES + cid

        _fill_rows(ones_v, CHUNK, 16, 0.0)
        for r in range(rpt // CHUNK):
            pltpu.sync_copy(ones_v, acc.at[pl.ds(sid * rpt + r * CHUNK, CHUNK)])
        _fill_rows(ones_v, CHUNK, 16, 1.0)
        plsc.subcore_barrier()

        base0 = wid * epw

        def chunk_body(ci, carry):
            base = base0 + ci * CHUNK
            pltpu.sync_copy(idx_hbm.at[pl.ds(base, CHUNK)], idx_v)
            pltpu.sync_copy(ones_v, acc.at[idx_v], add=True)
            return carry
        lax.fori_loop(0, nchunks, chunk_body, 0)

        plsc.subcore_barrier()
        for r in range(rpt // CHUNK):
            row0 = sid * rpt + r * CHUNK
            pltpu.sync_copy(acc.at[pl.ds(row0, CHUNK)],
                            out_hbm.at[cid, pl.ds(row0, CHUNK)])

    return count


@functools.cache
def _ptr_e_call(epad):
    epw = epad // NW
    nchunks = epw // CHUNK

    @functools.partial(
        pl.kernel,
        out_type=jax.ShapeDtypeStruct((epad,), f32),
        mesh=_sc_mesh(),
        scratch_types=[
            pltpu.VMEM((CHUNK,), i32),
            pltpu.VMEM((CHUNK,), i32),
            pltpu.VMEM((CHUNK, H), f32),
            pltpu.VMEM((CHUNK, H), f32),
            pltpu.VMEM((H,), f32),
            pltpu.VMEM((CHUNK,), f32),
            pltpu.SemaphoreType.DMA,
            pltpu.SemaphoreType.DMA,
        ],
    )
    def ptr_e(p_hbm, q_hbm, ce0_hbm, ce1_hbm, theta_hbm, out_hbm,
              idx0, idx1, pg, qg, th, ev, sem_a, sem_b):
        cid = lax.axis_index("c")
        sid = lax.axis_index("s")
        wid = sid * NCORES + cid
        pltpu.sync_copy(theta_hbm, th)
        lane = lax.iota(i32, 16)

        base0 = wid * epw

        def chunk_body(ci, carry):
            base = base0 + ci * CHUNK
            pltpu.sync_copy(ce0_hbm.at[pl.ds(base, CHUNK)], idx0)
            pltpu.sync_copy(ce1_hbm.at[pl.ds(base, CHUNK)], idx1)
            ca = pltpu.async_copy(p_hbm.at[idx0], pg, sem_a)
            cb = pltpu.async_copy(q_hbm.at[idx1], qg, sem_b)
            ca.wait()
            cb.wait()

            def group(g, carry2):
                def edge(t, evec):
                    e = g * 16 + t
                    acc = jnp.zeros((16,), f32)
                    for j in range(H // 16):
                        z = pg[e, pl.ds(j * 16, 16)] + qg[e, pl.ds(j * 16, 16)]
                        z = jnp.minimum(z, 15.0)
                        tz = 1.0 - 2.0 / (jnp.exp(2.0 * z) + 1.0)
                        acc = acc + th[pl.ds(j * 16, 16)] * tz
                    for sh in (8, 4, 2, 1):
                        acc = acc + acc.at[lane ^ sh].get(
                            mode="promise_in_bounds")
                    return jnp.where(lane == t, acc, evec)
                evec = lax.fori_loop(0, 16, edge, jnp.zeros((16,), f32))
                ev[pl.ds(g * 16, 16)] = evec
                return carry2
            lax.fori_loop(0, CHUNK // 16, group, 0)
            pltpu.sync_copy(ev, out_hbm.at[pl.ds(base, CHUNK)])
            return carry
        lax.fori_loop(0, nchunks, chunk_body, 0)

    return ptr_e


@functools.cache
def _ptr_scatter_call(epad):
    epw = epad // NW
    nchunks = epw // CHUNK
    rpt = NACC // NSUB

    @functools.partial(
        pl.kernel,
        out_type=jax.ShapeDtypeStruct((NCORES, NACC, H), f32),
        mesh=_sc_mesh(),
        scratch_types=[
            pltpu.VMEM((CHUNK,), i32),
            pltpu.VMEM((CHUNK,), i32),
            pltpu.VMEM((CHUNK, H), f32),
            pltpu.VMEM((CHUNK,), f32),
            pltpu.VMEM_SHARED((NACC, H), f32),
            pltpu.SemaphoreType.DMA,
        ],
    )
    def ptr_scatter(x_hbm, w_hbm, ce0_hbm, ce1_hbm, out_hbm,
                    idx0, idx1, xg, wv, acc, sem_a):
        cid = lax.axis_index("c")
        sid = lax.axis_index("s")
        wid = sid * NCORES + cid

        _fill_rows(xg, CHUNK, H, 0.0)
        for r in range(rpt // CHUNK):
            pltpu.sync_copy(xg, acc.at[pl.ds(sid * rpt + r * CHUNK, CHUNK)])
        plsc.subcore_barrier()

        base0 = wid * epw

        def chunk_body(ci, carry):
            base = base0 + ci * CHUNK
            pltpu.sync_copy(ce0_hbm.at[pl.ds(base, CHUNK)], idx0)
            pltpu.sync_copy(ce1_hbm.at[pl.ds(base, CHUNK)], idx1)
            pltpu.sync_copy(w_hbm.at[pl.ds(base, CHUNK)], wv)
            pltpu.async_copy(x_hbm.at[idx0], xg, sem_a).wait()

            def group(g, c2):
                wvec = wv[pl.ds(g * 16, 16)]

                def edge(t, c3):
                    e = g * 16 + t
                    ws = wvec.at[jnp.full((16,), t, i32)].get(
                        mode="promise_in_bounds")
                    for j in range(H // 16):
                        xg[e, pl.ds(j * 16, 16)] = xg[e, pl.ds(j * 16, 16)] * ws
                    return c3
                return lax.fori_loop(0, 16, edge, c2)
            lax.fori_loop(0, CHUNK // 16, group, 0)
            pltpu.sync_copy(xg, acc.at[idx1], add=True)
            return carry
        lax.fori_loop(0, nchunks, chunk_body, 0)

        plsc.subcore_barrier()
        for r in range(rpt // CHUNK):
            row0 = sid * rpt + r * CHUNK
            pltpu.sync_copy(acc.at[pl.ds(row0, CHUNK)],
                            out_hbm.at[cid, pl.ds(row0, CHUNK)])

    return ptr_scatter


def _sc_edge_msg(epad, a_pad, b_pad, dst_pad, src_pad):
    return _edge_msg_call(epad)(a_pad, b_pad, dst_pad, src_pad)


def _sc_count(epad, idx_pad):
    return _count_call(epad)(idx_pad)


def _sc_ptr_e(epad, p, q, ce0, ce1, theta):
    return _ptr_e_call(epad)(p, q, ce0, ce1, theta)


def _sc_ptr_scatter(epad, x_pad, w_pad, ce0, ce1):
    return _ptr_scatter_call(epad)(x_pad, w_pad, ce0, ce1)


def _rspec(k):
    return pl.BlockSpec((BLK, k), lambda i: (i, 0))


def _fspec(shape):
    return pl.BlockSpec(shape, lambda i: tuple(0 for _ in shape))


def _ispec():
    return pl.BlockSpec((1, 1, BLK), lambda i: (i, 0, 0))


def _dot(a, b):
    return jnp.dot(a, b, preferred_element_type=f32)


def _lrelu(t):
    return jnp.maximum(t, 0.01 * t)


def _enc2(x1, x2, w1, w2, b):
    k1 = x1.shape[1]
    k2 = x2.shape[1]

    def body(x1_ref, x2_ref, w1_ref, w2_ref, b_ref, o_ref):
        t = _dot(x1_ref[...], w1_ref[...]) + _dot(x2_ref[...], w2_ref[...]) + b_ref[...]
        o_ref[...] = _lrelu(t)

    return pl.pallas_call(
        body, grid=(NBLK,),
        in_specs=[_rspec(k1), _rspec(k2), _fspec((k1, H)), _fspec((k2, H)),
                  _fspec((1, H))],
        out_specs=_rspec(H),
        out_shape=jax.ShapeDtypeStruct((N_P, H), f32),
    )(x1, x2, w1, w2, b.reshape(1, H))


def _venc(vx, vn, lvl3, pe, w1, w2, b):
    def body(x1_ref, x2_ref, l_ref, pe_ref, w1_ref, w2_ref, b_ref, o_ref):
        t = _dot(x1_ref[...], w1_ref[...]) + _dot(x2_ref[...], w2_ref[...]) + b_ref[...]
        lv = l_ref[...].reshape(BLK)
        oh = (lv[:, None] == lax.broadcasted_iota(i32, (BLK, 100), 1)).astype(f32)
        o_ref[...] = _lrelu(t) + _dot(oh, pe_ref[...])

    return pl.pallas_call(
        body, grid=(NBLK,),
        in_specs=[_rspec(128), _rspec(32), _ispec(), _fspec((100, H)),
                  _fspec((128, H)), _fspec((32, H)), _fspec((1, H))],
        out_specs=_rspec(H),
        out_shape=jax.ShapeDtypeStruct((N_V, H), f32),
    )(vx, vn, lvl3, pe, w1, w2, b.reshape(1, H))


def _pe_kernel():
    def body(o_ref):
        pos = lax.broadcasted_iota(i32, (100, H), 0).astype(f32)
        col = lax.broadcasted_iota(i32, (100, H), 1)
        i2 = ((col // 2) * 2).astype(f32)
        denom = jnp.exp(i2 * (jnp.log(10000.0) / H))
        ang = pos / denom
        even = (col % 2) == 0
        o_ref[...] = jnp.where(even, jnp.sin(ang), jnp.cos(ang))

    return pl.pallas_call(
        body, grid=(1,),
        in_specs=[],
        out_specs=_fspec((100, H)),
        out_shape=jax.ShapeDtypeStruct((100, H), f32),
    )()


def _pre2(x, wd, ws, b):
    def body(x_ref, wd_ref, ws_ref, b_ref, a_ref, b2_ref):
        xv = x_ref[...]
        a_ref[...] = _dot(xv, wd_ref[...]) + b_ref[...]
        b2_ref[...] = _dot(xv, ws_ref[...])

    return pl.pallas_call(
        body, grid=(NBLK,),
        in_specs=[_rspec(H), _fspec((H, H)), _fspec((H, H)), _fspec((1, H))],
        out_specs=[_rspec(H), _rspec(H)],
        out_shape=[jax.ShapeDtypeStruct((N_P, H), f32),
                   jax.ShapeDtypeStruct((N_P, H), f32)],
    )(x, wd, ws, b.reshape(1, H))


def _vpre(v, lvl3, pe, w1, w2, w3, b):
    def body(v_ref, l_ref, pe_ref, w1_ref, w2_ref, w3_ref, b_ref, a_ref, b2_ref):
        vv = v_ref[...]
        d = _dot(pe_ref[...], w3_ref[...])
        lv = l_ref[...].reshape(BLK)
        oh = (lv[:, None] == lax.broadcasted_iota(i32, (BLK, 100), 1)).astype(f32)
        posd = _dot(oh, d)
        a_ref[...] = _dot(vv, w1_ref[...]) + posd + b_ref[...]
        b2_ref[...] = _dot(vv, w2_ref[...]) - posd

    return pl.pallas_call(
        body, grid=(NBLK,),
        in_specs=[_rspec(H), _ispec(), _fspec((100, H)), _fspec((H, H)),
                  _fspec((H, H)), _fspec((H, H)), _fspec((1, H))],
        out_specs=[_rspec(H), _rspec(H)],
        out_shape=[jax.ShapeDtypeStruct((N_V, H), f32),
                   jax.ShapeDtypeStruct((N_V, H), f32)],
    )(v, lvl3, pe, w1, w2, w3, b.reshape(1, H))


def _csum(cl3, x):
    def body(c_ref, x_ref, s_ref, n_ref):
        cv = c_ref[...].reshape(BLK)
        oh = (cv[:, None] == lax.broadcasted_iota(i32, (BLK, N_C), 1)).astype(f32)

        @pl.when(pl.program_id(0) == 0)
        def _():
            s_ref[...] = jnp.zeros_like(s_ref)
            n_ref[...] = jnp.zeros_like(n_ref)

        s_ref[...] += lax.dot_general(oh, x_ref[...], (((0,), (0,)), ((), ())),
                                      preferred_element_type=f32)
        n_ref[...] += jnp.sum(oh, axis=0)[:, None]

    return pl.pallas_call(
        body, grid=(NBLK,),
        in_specs=[_ispec(), _rspec(H)],
        out_specs=[_fspec((N_C, H)), _fspec((N_C, 1))],
        out_shape=[jax.ShapeDtypeStruct((N_C, H), f32),
                   jax.ShapeDtypeStruct((N_C, 1), f32)],
    )(cl3, x)


def _cmw3(csum, ccnt, wu3):
    def body(s_ref, n_ref, w_ref, o_ref):
        cm = s_ref[...] / jnp.maximum(n_ref[...], 1.0)
        o_ref[...] = _dot(cm, w_ref[...])

    return pl.pallas_call(
        body, grid=(1,),
        in_specs=[_fspec((N_C, H)), _fspec((N_C, 1)), _fspec((H, H))],
        out_specs=_fspec((N_C, H)),
        out_shape=jax.ShapeDtypeStruct((N_C, H), f32),
    )(csum, ccnt, wu3)


def _pupd(x, m0, m1, d0, d1, cl3, cmw3, ratio, wu1, wu2, bu):
    def body(x_ref, m0_ref, m1_ref, d0_ref, d1_ref, c_ref, cm_ref, r_ref,
             w1_ref, w2_ref, b_ref, o_ref):
        xv = x_ref[...]
        deg = jnp.maximum(d0_ref[...][:, :1] + d1_ref[...][:, :1], 1.0)
        aggr = (m0_ref[...] + m1_ref[...]) / deg
        cv = c_ref[...].reshape(BLK)
        oh = (cv[:, None] == lax.broadcasted_iota(i32, (BLK, N_C), 1)).astype(f32)
        cterm = r_ref[...] * _dot(oh, cm_ref[...])
        t = _dot(xv, w1_ref[...]) + _dot(aggr, w2_ref[...]) + cterm + b_ref[...]
        o_ref[...] = xv + _lrelu(t)

    return pl.pallas_call(
        body, grid=(NBLK,),
        in_specs=[_rspec(H), _rspec(H), _rspec(H), _rspec(16), _rspec(16),
                  _ispec(), _fspec((N_C, H)), _rspec(1),
                  _fspec((H, H)), _fspec((H, H)), _fspec((1, H))],
        out_specs=_rspec(H),
        out_shape=jax.ShapeDtypeStruct((N_P, H), f32),
    )(x, m0, m1, d0, d1, cl3, cmw3, ratio, wu1, wu2, bu.reshape(1, H))


def _vupd(v, m0, m1, wu1, wu2, bu):
    def body(v_ref, m0_ref, m1_ref, w1_ref, w2_ref, b_ref, o_ref):
        vv = v_ref[...]
        aggr = m0_ref[...] + m1_ref[...]
        t = _dot(vv, w1_ref[...]) + _dot(aggr, w2_ref[...]) + b_ref[...]
        o_ref[...] = vv + _lrelu(t)

    return pl.pallas_call(
        body, grid=(NBLK,),
        in_specs=[_rspec(H), _rspec(H), _rspec(H),
                  _fspec((H, H)), _fspec((H, H)), _fspec((1, H))],
        out_specs=_rspec(H),
        out_shape=jax.ShapeDtypeStruct((N_V, H), f32),
    )(v, m0, m1, wu1, wu2, bu.reshape(1, H))


def _ptrprep(x, v, m1w, m1b, m2w, m2b, wp, bsum, wv):
    def body(x_ref, v_ref, m1w_ref, m1b_ref, m2w_ref, m2b_ref, wp_ref,
             bs_ref, wv_ref, mask_ref, p_ref, q_ref):
        vv = v_ref[...]
        h = _lrelu(_dot(vv, m1w_ref[...]) + m1b_ref[...])
        mask_ref[...] = jax.nn.sigmoid(_dot(h, m2w_ref[...]) + m2b_ref[...])
        p_ref[...] = _dot(x_ref[...], wp_ref[...]) + bs_ref[...]
        q_ref[...] = _dot(vv, wv_ref[...])

    return pl.pallas_call(
        body, grid=(NBLK,),
        in_specs=[_rspec(H), _rspec(H), _fspec((H, H)), _fspec((1, H)),
                  _fspec((H, 1)), _fspec((1, 1)), _fspec((H, H)),
                  _fspec((1, H)), _fspec((H, H))],
        out_specs=[_rspec(1), _rspec(H), _rspec(H)],
        out_shape=[jax.ShapeDtypeStruct((N_V, 1), f32),
                   jax.ShapeDtypeStruct((N_P, H), f32),
                   jax.ShapeDtypeStruct((N_V, H), f32)],
    )(x, v, m1w, m1b.reshape(1, H), m2w, m2b.reshape(1, 1), wp,
      bsum.reshape(1, H), wv)


def _softmax_w(e2d, g2d):
    rows = E_C // H

    def body(e_ref, g_ref, w_ref, s_ref):
        s = e_ref[...] + g_ref[...]
        w = jnp.exp(s - jnp.max(s))
        w_ref[...] = w
        s_ref[...] = jnp.sum(w).reshape(1, 1)

    return pl.pallas_call(
        body, grid=(1,),
        in_specs=[_fspec((rows, H))] * 2,
        out_specs=[_fspec((rows, H)), _fspec((1, 1))],
        out_shape=[jax.ShapeDtypeStruct((rows, H), f32),
                   jax.ShapeDtypeStruct((1, 1), f32)],
    )(e2d, g2d)


def _ptr_final(v, mask, s0, s1, ssum):
    def body(v_ref, k_ref, s0_ref, s1_ref, ss_ref, o_ref):
        o_ref[...] = v_ref[...] + k_ref[...] * (
            (s0_ref[...] + s1_ref[...]) / ss_ref[0, 0])

    return pl.pallas_call(
        body, grid=(NBLK,),
        in_specs=[_rspec(H), _rspec(1), _rspec(H), _rspec(H), _fspec((1, 1))],
        out_specs=_rspec(H),
        out_shape=jax.ShapeDtypeStruct((N_V, H), f32),
    )(v, mask, s0, s1, ssum)


def _pad_idx(idx, epad, fill):
    return jnp.concatenate([idx, jnp.full((epad - idx.shape[0],), fill, i32)])


def _pad_rows(t):
    return jnp.concatenate([t, jnp.zeros((NACC - t.shape[0], H), f32)], axis=0)


def kernel(local_x, local_edge_index, node_cluster, node_ratio, voxel_x,
           voxel_edge_index, voxel_level, cross_edge_index, program_noise,
           voxel_noise, params):
    psrc = _pad_idx(local_edge_index[0], EP_PAD, 0)
    pdst = _pad_idx(local_edge_index[1], EP_PAD, N_P)
    vsrc = _pad_idx(voxel_edge_index[0], EV_PAD, 0)
    vdst = _pad_idx(voxel_edge_index[1], EV_PAD, N_V)
    ce0 = _pad_idx(cross_edge_index[0], EC_PAD, 0)
    ce1g = _pad_idx(cross_edge_index[1], EC_PAD, 0)
    ce1s = _pad_idx(cross_edge_index[1], EC_PAD, N_V)
    cl3 = node_cluster.reshape(NBLK, 1, BLK)
    lvl3 = voxel_level.reshape(NBLK, 1, BLK)

    gumbel = {}
    for li in (1, 3):
        u = jax.random.uniform(jax.random.fold_in(jax.random.key(42), li),
                               (E_C,), minval=1e-9, maxval=1.0, dtype=f32)
        gumbel[li] = (-jnp.log(-jnp.log(u))).reshape(E_C // H, H)

    deg = _sc_count(EP_PAD, pdst)
    d0 = deg[0][:N_P]
    d1 = deg[1][:N_P]

    pw = params["p_enc"]["W"]
    x = _enc2(local_x, program_noise, pw[:128], pw[128:], params["p_enc"]["b"])
    for l in range(P_STEPS):
        wm = params["p_msg"][l]["W"]
        a, b = _pre2(x, wm[:H], wm[H:], params["p_msg"][l]["b"])
        ms = _sc_edge_msg(EP_PAD, _pad_rows(a), _pad_rows(b), pdst, psrc)
        cs, cn = _csum(cl3, x)
        wu = params["p_upd"][l]["W"]
        cm3 = _cmw3(cs, cn, wu[2 * H:])
        x = _pupd(x, ms[0][:N_P], ms[1][:N_P], d0, d1, cl3, cm3, node_ratio,
                  wu[:H], wu[H:2 * H], params["p_upd"][l]["b"])

    pe = _pe_kernel()
    vw = params["v_enc"]["W"]
    v = _venc(voxel_x, voxel_noise, lvl3, pe, vw[:128], vw[128:],
              params["v_enc"]["b"])
    xpad = _pad_rows(x)
    p = params["ptr"]
    for li in range(V_STEPS):
        wm = params["v_msg"][li]["W"]
        a, b = _vpre(v, lvl3, pe, wm[:H], wm[H:2 * H], wm[2 * H:],
                     params["v_msg"][li]["b"])
        ms = _sc_edge_msg(EV_PAD, _pad_rows(a), _pad_rows(b), vdst, vsrc)
        wu = params["v_upd"][li]["W"]
        v = _vupd(v, ms[0][:N_V], ms[1][:N_V], wu[:H], wu[H:],
                  params["v_upd"][li]["b"])
        if (li + 1) % 2 == 0:
            mask, pp, qq = _ptrprep(
                x, v, p["m1"]["W"], p["m1"]["b"], p["m2"]["W"], p["m2"]["b"],
                p["Wp"]["W"], p["Wp"]["b"] + p["Wv"]["b"], p["Wv"]["W"])
            e_edge = _sc_ptr_e(EC_PAD, pp, qq, ce0, ce1g, p["theta"][:, 0])
            e2d = e_edge[:E_C].reshape(E_C // H, H)
            w2d, ssum = _softmax_w(e2d, gumbel[li])
            wpad = jnp.concatenate(
                [w2d.reshape(E_C), jnp.zeros((EC_PAD - E_C,), f32)])
            sc = _sc_ptr_scatter(EC_PAD, xpad, wpad, ce0, ce1s)
            v = _ptr_final(v, mask, sc[0][:N_V], sc[1][:N_V], ssum)
    return v


# speedup vs baseline: 2.9318x; 1.3587x over previous
import functools

import jax
import jax.numpy as jnp
from jax import lax
from jax.experimental import pallas as pl
from jax.experimental.pallas import tpu as pltpu
from jax.experimental.pallas import tpu_sc as plsc

H = 128
N_P = 10000
N_V = 10000
N_C = 500
P_STEPS = 3
V_STEPS = 4
E_P = 160000
E_V = 320000
E_C = 320000

NCORES = 2
NSUB = 16
NW = NCORES * NSUB
CHUNK = 128
NACC = 10240

EP_PAD = 163840
EV_PAD = 327680
EC_PAD = 327680

BLK = 400
NBLK = 25

f32 = jnp.float32
i32 = jnp.int32


def _sc_mesh():
    return plsc.VectorSubcoreMesh(core_axis_name="c", subcore_axis_name="s")


def _fill_rows(buf, nrows, width, val):
    def row(e, carry):
        for j in range(width // 16):
            buf[e, pl.ds(j * 16, 16)] = jnp.full((16,), val, f32)
        return carry
    lax.fori_loop(0, nrows, row, 0)


@functools.cache
def _edge_msg_call(epad):
    ck = CHUNK // 2
    epw = epad // NW
    nchunks = epw // ck
    rpt = NACC // NSUB

    @functools.partial(
        pl.kernel,
        out_type=jax.ShapeDtypeStruct((NCORES, NACC, H), f32),
        mesh=_sc_mesh(),
        scratch_types=[
            [pltpu.VMEM((ck,), i32)] * 2,
            [pltpu.VMEM((ck,), i32)] * 2,
            [pltpu.VMEM((ck, H), f32)] * 2,
            [pltpu.VMEM((ck, H), f32)] * 2,
            pltpu.VMEM_SHARED((NACC, H), f32),
            [pltpu.SemaphoreType.DMA] * 2,
            [pltpu.SemaphoreType.DMA] * 2,
        ],
    )
    def edge_msg(a_hbm, b_hbm, dst_hbm, src_hbm, out_hbm,
                 idx_d, idx_s, ag, bg, acc, sem_a, sem_b):
        cid = lax.axis_index("c")
        sid = lax.axis_index("s")
        wid = sid * NCORES + cid

        _fill_rows(ag[0], ck, H, 0.0)
        for r in range(rpt // ck):
            pltpu.sync_copy(ag[0], acc.at[pl.ds(sid * rpt + r * ck, ck)])
        plsc.subcore_barrier()

        base0 = wid * epw

        def idx_load(c, par):
            pltpu.sync_copy(dst_hbm.at[pl.ds(base0 + c * ck, ck)],
                            idx_d[par])
            pltpu.sync_copy(src_hbm.at[pl.ds(base0 + c * ck, ck)],
                            idx_s[par])

        def gather_start(par):
            pltpu.async_copy(a_hbm.at[idx_d[par]], ag[par], sem_a[par])
            pltpu.async_copy(b_hbm.at[idx_s[par]], bg[par], sem_b[par])

        def gather_wait(par):
            pltpu.make_async_copy(a_hbm.at[idx_d[par]], ag[par],
                                  sem_a[par]).wait()
            pltpu.make_async_copy(b_hbm.at[idx_s[par]], bg[par],
                                  sem_b[par]).wait()

        idx_load(0, 0)
        gather_start(0)

        def body(k2, carry):
            c0 = k2 * 2
            for par in (0, 1):
                c = c0 + par
                idx_load(c + 1, par ^ 1)
                gather_start(par ^ 1)
                gather_wait(par)

                def edge(e, c2):
                    for j in range(H // 16):
                        z = (ag[par][e, pl.ds(j * 16, 16)]
                             + bg[par][e, pl.ds(j * 16, 16)])
                        ag[par][e, pl.ds(j * 16, 16)] = jnp.maximum(z, 0.01 * z)
                    return c2
                lax.fori_loop(0, ck, edge, 0)
                pltpu.sync_copy(ag[par], acc.at[idx_d[par]], add=True)
            return carry
        lax.fori_loop(0, nchunks // 2, body, 0)

        gather_wait(0)

        plsc.subcore_barrier()
        for r in range(rpt // CHUNK):
            row0 = sid * rpt + r * CHUNK
            pltpu.sync_copy(acc.at[pl.ds(row0, CHUNK)],
                            out_hbm.at[cid, pl.ds(row0, CHUNK)])

    return edge_msg


@functools.cache
def _count_call(epad):
    epw = epad // NW
    nchunks = epw // CHUNK
    rpt = NACC // NSUB

    @functools.partial(
        pl.kernel,
        out_type=jax.ShapeDtypeStruct((NCORES, NACC, 16), f32),
        mesh=_sc_mesh(),
        scratch_types=[
            pltpu.VMEM((CHUNK,), i32),
            pltpu.VMEM((CHUNK, 16), f32),
            pltpu.VMEM_SHARED((NACC, 16), f32),
        ],
    )
    def count(idx_hbm, out_hbm, idx_v, ones_v, acc):
        cid = lax.axis_index("c")
        sid = lax.axis_index("s")
        wid = sid * NCORES + cid

        _fill_rows(ones_v, CHUNK, 16, 0.0)
        for r in range(rpt // CHUNK):
            pltpu.sync_copy(ones_v, acc.at[pl.ds(sid * rpt + r * CHUNK, CHUNK)])
        _fill_rows(ones_v, CHUNK, 16, 1.0)
        plsc.subcore_barrier()

        base0 = wid * epw

        def chunk_body(ci, carry):
            base = base0 + ci * CHUNK
            pltpu.sync_copy(idx_hbm.at[pl.ds(base, CHUNK)], idx_v)
            pltpu.sync_copy(ones_v, acc.at[idx_v], add=True)
            return carry
        lax.fori_loop(0, nchunks, chunk_body, 0)

        plsc.subcore_barrier()
        for r in range(rpt // CHUNK):
            row0 = sid * rpt + r * CHUNK
            pltpu.sync_copy(acc.at[pl.ds(row0, CHUNK)],
                            out_hbm.at[cid, pl.ds(row0, CHUNK)])

    return count


@functools.cache
def _ptr_e_call(epad):
    epw = epad // NW
    nchunks = epw // CHUNK

    @functools.partial(
        pl.kernel,
        out_type=jax.ShapeDtypeStruct((epad,), f32),
        mesh=_sc_mesh(),
        scratch_types=[
            [pltpu.VMEM((CHUNK,), i32)] * 2,
            [pltpu.VMEM((CHUNK,), i32)] * 2,
            [pltpu.VMEM((CHUNK, H), f32)] * 2,
            [pltpu.VMEM((CHUNK, H), f32)] * 2,
            pltpu.VMEM((H,), f32),
            pltpu.VMEM((CHUNK,), f32),
            [pltpu.SemaphoreType.DMA] * 2,
            [pltpu.SemaphoreType.DMA] * 2,
        ],
    )
    def ptr_e(p_hbm, q_hbm, ce0_hbm, ce1_hbm, theta_hbm, out_hbm,
              idx0, idx1, pg, qg, th, ev, sem_a, sem_b):
        cid = lax.axis_index("c")
        sid = lax.axis_index("s")
        wid = sid * NCORES + cid
        pltpu.sync_copy(theta_hbm, th)
        lane = lax.iota(i32, 16)

        base0 = wid * epw

        def idx_load(c, par):
            pltpu.sync_copy(ce0_hbm.at[pl.ds(base0 + c * CHUNK, CHUNK)],
                            idx0[par])
            pltpu.sync_copy(ce1_hbm.at[pl.ds(base0 + c * CHUNK, CHUNK)],
                            idx1[par])

        def gather_start(par):
            pltpu.async_copy(p_hbm.at[idx0[par]], pg[par], sem_a[par])
            pltpu.async_copy(q_hbm.at[idx1[par]], qg[par], sem_b[par])

        def gather_wait(par):
            pltpu.make_async_copy(p_hbm.at[idx0[par]], pg[par],
                                  sem_a[par]).wait()
            pltpu.make_async_copy(q_hbm.at[idx1[par]], qg[par],
                                  sem_b[par]).wait()

        idx_load(0, 0)
        gather_start(0)

        def body(k2, carry):
            c0 = k2 * 2
            for par in (0, 1):
                c = c0 + par
                idx_load(c + 1, par ^ 1)
                gather_start(par ^ 1)
                gather_wait(par)

                def group(g, carry2):
                    def edge(t, evec):
                        e = g * 16 + t
                        acc = jnp.zeros((16,), f32)
                        for j in range(H // 16):
                            z = (pg[par][e, pl.ds(j * 16, 16)]
                                 + qg[par][e, pl.ds(j * 16, 16)])
                            z = jnp.minimum(z, 15.0)
                            tz = 1.0 - 2.0 / (jnp.exp(2.0 * z) + 1.0)
                            acc = acc + th[pl.ds(j * 16, 16)] * tz
                        for sh in (8, 4, 2, 1):
                            acc = acc + acc.at[lane ^ sh].get(
                                mode="promise_in_bounds")
                        return jnp.where(lane == t, acc, evec)
                    evec = lax.fori_loop(0, 16, edge, jnp.zeros((16,), f32))
                    ev[pl.ds(g * 16, 16)] = evec
                    return carry2
                lax.fori_loop(0, CHUNK // 16, group, 0)
                pltpu.sync_copy(ev, out_hbm.at[pl.ds(base0 + c * CHUNK, CHUNK)])
            return carry
        lax.fori_loop(0, nchunks // 2, body, 0)

        gather_wait(0)

    return ptr_e


@functools.cache
def _ptr_scatter_call(epad):
    epw = epad // NW
    nchunks = epw // CHUNK
    rpt = NACC // NSUB

    @functools.partial(
        pl.kernel,
        out_type=jax.ShapeDtypeStruct((NCORES, NACC, H), f32),
        mesh=_sc_mesh(),
        scratch_types=[
            [pltpu.VMEM((CHUNK,), i32)] * 2,
            [pltpu.VMEM((CHUNK,), i32)] * 2,
            [pltpu.VMEM((CHUNK, H), f32)] * 2,
            [pltpu.VMEM((CHUNK,), f32)] * 2,
            pltpu.VMEM_SHARED((NACC, H), f32),
            [pltpu.SemaphoreType.DMA] * 2,
        ],
    )
    def ptr_scatter(x_hbm, w_hbm, ce0_hbm, ce1_hbm, out_hbm,
                    idx0, idx1, xg, wv, acc, sem_a):
        cid = lax.axis_index("c")
        sid = lax.axis_index("s")
        wid = sid * NCORES + cid

        _fill_rows(xg[0], CHUNK, H, 0.0)
        for r in range(rpt // CHUNK):
            pltpu.sync_copy(xg[0], acc.at[pl.ds(sid * rpt + r * CHUNK, CHUNK)])
        plsc.subcore_barrier()

        base0 = wid * epw

        def idx_load(c, par):
            pltpu.sync_copy(ce0_hbm.at[pl.ds(base0 + c * CHUNK, CHUNK)],
                            idx0[par])
            pltpu.sync_copy(ce1_hbm.at[pl.ds(base0 + c * CHUNK, CHUNK)],
                            idx1[par])
            pltpu.sync_copy(w_hbm.at[pl.ds(base0 + c * CHUNK, CHUNK)],
                            wv[par])

        def gather_start(par):
            pltpu.async_copy(x_hbm.at[idx0[par]], xg[par], sem_a[par])

        def gather_wait(par):
            pltpu.make_async_copy(x_hbm.at[idx0[par]], xg[par],
                                  sem_a[par]).wait()

        idx_load(0, 0)
        gather_start(0)

        def body(k2, carry):
            c0 = k2 * 2
            for par in (0, 1):
                c = c0 + par
                idx_load(c + 1, par ^ 1)
                gather_start(par ^ 1)
                gather_wait(par)

                def group(g, c2):
                    wvec = wv[par][pl.ds(g * 16, 16)]

                    def edge(t, c3):
                        e = g * 16 + t
                        ws = wvec.at[jnp.full((16,), t, i32)].get(
                            mode="promise_in_bounds")
                        for j in range(H // 16):
                            xg[par][e, pl.ds(j * 16, 16)] = (
                                xg[par][e, pl.ds(j * 16, 16)] * ws)
                        return c3
                    return lax.fori_loop(0, 16, edge, c2)
                lax.fori_loop(0, CHUNK // 16, group, 0)
                pltpu.sync_copy(xg[par], acc.at[idx1[par]], add=True)
            return carry
        lax.fori_loop(0, nchunks // 2, body, 0)

        gather_wait(0)

        plsc.subcore_barrier()
        for r in range(rpt // CHUNK):
            row0 = sid * rpt + r * CHUNK
            pltpu.sync_copy(acc.at[pl.ds(row0, CHUNK)],
                            out_hbm.at[cid, pl.ds(row0, CHUNK)])

    return ptr_scatter


def _sc_edge_msg(epad, a_pad, b_pad, dst_pad, src_pad):
    return _edge_msg_call(epad)(a_pad, b_pad, dst_pad, src_pad)


def _sc_count(epad, idx_pad):
    return _count_call(epad)(idx_pad)


def _sc_ptr_e(epad, p, q, ce0, ce1, theta):
    return _ptr_e_call(epad)(p, q, ce0, ce1, theta)


def _sc_ptr_scatter(epad, x_pad, w_pad, ce0, ce1):
    return _ptr_scatter_call(epad)(x_pad, w_pad, ce0, ce1)


def _rspec(k):
    return pl.BlockSpec((BLK, k), lambda i: (i, 0))


def _fspec(shape):
    return pl.BlockSpec(shape, lambda i: tuple(0 for _ in shape))


def _ispec():
    return pl.BlockSpec((1, 1, BLK), lambda i: (i, 0, 0))


def _dot(a, b):
    return jnp.dot(a, b, preferred_element_type=f32)


def _lrelu(t):
    return jnp.maximum(t, 0.01 * t)


def _enc2(x1, x2, w1, w2, b):
    k1 = x1.shape[1]
    k2 = x2.shape[1]

    def body(x1_ref, x2_ref, w1_ref, w2_ref, b_ref, o_ref):
        t = _dot(x1_ref[...], w1_ref[...]) + _dot(x2_ref[...], w2_ref[...]) + b_ref[...]
        o_ref[...] = _lrelu(t)

    return pl.pallas_call(
        body, grid=(NBLK,),
        in_specs=[_rspec(k1), _rspec(k2), _fspec((k1, H)), _fspec((k2, H)),
                  _fspec((1, H))],
        out_specs=_rspec(H),
        out_shape=jax.ShapeDtypeStruct((N_P, H), f32),
    )(x1, x2, w1, w2, b.reshape(1, H))


def _venc(vx, vn, lvl3, pe, w1, w2, b):
    def body(x1_ref, x2_ref, l_ref, pe_ref, w1_ref, w2_ref, b_ref, o_ref):
        t = _dot(x1_ref[...], w1_ref[...]) + _dot(x2_ref[...], w2_ref[...]) + b_ref[...]
        lv = l_ref[...].reshape(BLK)
        oh = (lv[:, None] == lax.broadcasted_iota(i32, (BLK, 100), 1)).astype(f32)
        o_ref[...] = _lrelu(t) + _dot(oh, pe_ref[...])

    return pl.pallas_call(
        body, grid=(NBLK,),
        in_specs=[_rspec(128), _rspec(32), _ispec(), _fspec((100, H)),
                  _fspec((128, H)), _fspec((32, H)), _fspec((1, H))],
        out_specs=_rspec(H),
        out_shape=jax.ShapeDtypeStruct((N_V, H), f32),
    )(vx, vn, lvl3, pe, w1, w2, b.reshape(1, H))


def _pe_kernel():
    def body(o_ref):
        pos = lax.broadcasted_iota(i32, (100, H), 0).astype(f32)
        col = lax.broadcasted_iota(i32, (100, H), 1)
        i2 = ((col // 2) * 2).astype(f32)
        denom = jnp.exp(i2 * (jnp.log(10000.0) / H))
        ang = pos / denom
        even = (col % 2) == 0
        o_ref[...] = jnp.where(even, jnp.sin(ang), jnp.cos(ang))

    return pl.pallas_call(
        body, grid=(1,),
        in_specs=[],
        out_specs=_fspec((100, H)),
        out_shape=jax.ShapeDtypeStruct((100, H), f32),
    )()


def _pre2(x, wd, ws, b):
    def body(x_ref, wd_ref, ws_ref, b_ref, a_ref, b2_ref):
        xv = x_ref[...]
        a_ref[...] = _dot(xv, wd_ref[...]) + b_ref[...]
        b2_ref[...] = _dot(xv, ws_ref[...])

    return pl.pallas_call(
        body, grid=(NBLK,),
        in_specs=[_rspec(H), _fspec((H, H)), _fspec((H, H)), _fspec((1, H))],
        out_specs=[_rspec(H), _rspec(H)],
        out_shape=[jax.ShapeDtypeStruct((N_P, H), f32),
                   jax.ShapeDtypeStruct((N_P, H), f32)],
    )(x, wd, ws, b.reshape(1, H))


def _vpre(v, lvl3, pe, w1, w2, w3, b):
    def body(v_ref, l_ref, pe_ref, w1_ref, w2_ref, w3_ref, b_ref, a_ref, b2_ref):
        vv = v_ref[...]
        d = _dot(pe_ref[...], w3_ref[...])
        lv = l_ref[...].reshape(BLK)
        oh = (lv[:, None] == lax.broadcasted_iota(i32, (BLK, 100), 1)).astype(f32)
        posd = _dot(oh, d)
        a_ref[...] = _dot(vv, w1_ref[...]) + posd + b_ref[...]
        b2_ref[...] = _dot(vv, w2_ref[...]) - posd

    return pl.pallas_call(
        body, grid=(NBLK,),
        in_specs=[_rspec(H), _ispec(), _fspec((100, H)), _fspec((H, H)),
                  _fspec((H, H)), _fspec((H, H)), _fspec((1, H))],
        out_specs=[_rspec(H), _rspec(H)],
        out_shape=[jax.ShapeDtypeStruct((N_V, H), f32),
                   jax.ShapeDtypeStruct((N_V, H), f32)],
    )(v, lvl3, pe, w1, w2, w3, b.reshape(1, H))


def _csum(cl3, x):
    def body(c_ref, x_ref, s_ref, n_ref):
        cv = c_ref[...].reshape(BLK)
        oh = (cv[:, None] == lax.broadcasted_iota(i32, (BLK, N_C), 1)).astype(f32)

        @pl.when(pl.program_id(0) == 0)
        def _():
            s_ref[...] = jnp.zeros_like(s_ref)
            n_ref[...] = jnp.zeros_like(n_ref)

        s_ref[...] += lax.dot_general(oh, x_ref[...], (((0,), (0,)), ((), ())),
                                      preferred_element_type=f32)
        n_ref[...] += jnp.sum(oh, axis=0)[:, None]

    return pl.pallas_call(
        body, grid=(NBLK,),
        in_specs=[_ispec(), _rspec(H)],
        out_specs=[_fspec((N_C, H)), _fspec((N_C, 1))],
        out_shape=[jax.ShapeDtypeStruct((N_C, H), f32),
                   jax.ShapeDtypeStruct((N_C, 1), f32)],
    )(cl3, x)


def _cmw3(csum, ccnt, wu3):
    def body(s_ref, n_ref, w_ref, o_ref):
        cm = s_ref[...] / jnp.maximum(n_ref[...], 1.0)
        o_ref[...] = _dot(cm, w_ref[...])

    return pl.pallas_call(
        body, grid=(1,),
        in_specs=[_fspec((N_C, H)), _fspec((N_C, 1)), _fspec((H, H))],
        out_specs=_fspec((N_C, H)),
        out_shape=jax.ShapeDtypeStruct((N_C, H), f32),
    )(csum, ccnt, wu3)


def _pupd(x, m0, m1, d0, d1, cl3, cmw3, ratio, wu1, wu2, bu):
    def body(x_ref, m0_ref, m1_ref, d0_ref, d1_ref, c_ref, cm_ref, r_ref,
             w1_ref, w2_ref, b_ref, o_ref):
        xv = x_ref[...]
        deg = jnp.maximum(d0_ref[...][:, :1] + d1_ref[...][:, :1], 1.0)
        aggr = (m0_ref[...] + m1_ref[...]) / deg
        cv = c_ref[...].reshape(BLK)
        oh = (cv[:, None] == lax.broadcasted_iota(i32, (BLK, N_C), 1)).astype(f32)
        cterm = r_ref[...] * _dot(oh, cm_ref[...])
        t = _dot(xv, w1_ref[...]) + _dot(aggr, w2_ref[...]) + cterm + b_ref[...]
        o_ref[...] = xv + _lrelu(t)

    return pl.pallas_call(
        body, grid=(NBLK,),
        in_specs=[_rspec(H), _rspec(H), _rspec(H), _rspec(16), _rspec(16),
                  _ispec(), _fspec((N_C, H)), _rspec(1),
                  _fspec((H, H)), _fspec((H, H)), _fspec((1, H))],
        out_specs=_rspec(H),
        out_shape=jax.ShapeDtypeStruct((N_P, H), f32),
    )(x, m0, m1, d0, d1, cl3, cmw3, ratio, wu1, wu2, bu.reshape(1, H))


def _vupd(v, m0, m1, wu1, wu2, bu):
    def body(v_ref, m0_ref, m1_ref, w1_ref, w2_ref, b_ref, o_ref):
        vv = v_ref[...]
        aggr = m0_ref[...] + m1_ref[...]
        t = _dot(vv, w1_ref[...]) + _dot(aggr, w2_ref[...]) + b_ref[...]
        o_ref[...] = vv + _lrelu(t)

    return pl.pallas_call(
        body, grid=(NBLK,),
        in_specs=[_rspec(H), _rspec(H), _rspec(H),
                  _fspec((H, H)), _fspec((H, H)), _fspec((1, H))],
        out_specs=_rspec(H),
        out_shape=jax.ShapeDtypeStruct((N_V, H), f32),
    )(v, m0, m1, wu1, wu2, bu.reshape(1, H))


def _ptrprep(x, v, m1w, m1b, m2w, m2b, wp, bsum, wv):
    def body(x_ref, v_ref, m1w_ref, m1b_ref, m2w_ref, m2b_ref, wp_ref,
             bs_ref, wv_ref, mask_ref, p_ref, q_ref):
        vv = v_ref[...]
        h = _lrelu(_dot(vv, m1w_ref[...]) + m1b_ref[...])
        mask_ref[...] = jax.nn.sigmoid(_dot(h, m2w_ref[...]) + m2b_ref[...])
        p_ref[...] = _dot(x_ref[...], wp_ref[...]) + bs_ref[...]
        q_ref[...] = _dot(vv, wv_ref[...])

    return pl.pallas_call(
        body, grid=(NBLK,),
        in_specs=[_rspec(H), _rspec(H), _fspec((H, H)), _fspec((1, H)),
                  _fspec((H, 1)), _fspec((1, 1)), _fspec((H, H)),
                  _fspec((1, H)), _fspec((H, H))],
        out_specs=[_rspec(1), _rspec(H), _rspec(H)],
        out_shape=[jax.ShapeDtypeStruct((N_V, 1), f32),
                   jax.ShapeDtypeStruct((N_P, H), f32),
                   jax.ShapeDtypeStruct((N_V, H), f32)],
    )(x, v, m1w, m1b.reshape(1, H), m2w, m2b.reshape(1, 1), wp,
      bsum.reshape(1, H), wv)


def _softmax_w(e2d, g2d):
    rows = E_C // H

    def body(e_ref, g_ref, w_ref, s_ref):
        s = e_ref[...] + g_ref[...]
        w = jnp.exp(s - jnp.max(s))
        w_ref[...] = w
        s_ref[...] = jnp.sum(w).reshape(1, 1)

    return pl.pallas_call(
        body, grid=(1,),
        in_specs=[_fspec((rows, H))] * 2,
        out_specs=[_fspec((rows, H)), _fspec((1, 1))],
        out_shape=[jax.ShapeDtypeStruct((rows, H), f32),
                   jax.ShapeDtypeStruct((1, 1), f32)],
    )(e2d, g2d)


def _ptr_final(v, mask, s0, s1, ssum):
    def body(v_ref, k_ref, s0_ref, s1_ref, ss_ref, o_ref):
        o_ref[...] = v_ref[...] + k_ref[...] * (
            (s0_ref[...] + s1_ref[...]) / ss_ref[0, 0])

    return pl.pallas_call(
        body, grid=(NBLK,),
        in_specs=[_rspec(H), _rspec(1), _rspec(H), _rspec(H), _fspec((1, 1))],
        out_specs=_rspec(H),
        out_shape=jax.ShapeDtypeStruct((N_V, H), f32),
    )(v, mask, s0, s1, ssum)


def _pad_idx(idx, epad, fill):
    return jnp.concatenate([idx, jnp.full((epad - idx.shape[0],), fill, i32),
                            jnp.zeros((2 * CHUNK,), i32)])


def _pad_rows(t):
    return jnp.concatenate([t, jnp.zeros((NACC - t.shape[0], H), f32)], axis=0)


def kernel(local_x, local_edge_index, node_cluster, node_ratio, voxel_x,
           voxel_edge_index, voxel_level, cross_edge_index, program_noise,
           voxel_noise, params):
    psrc = _pad_idx(local_edge_index[0], EP_PAD, 0)
    pdst = _pad_idx(local_edge_index[1], EP_PAD, N_P)
    vsrc = _pad_idx(voxel_edge_index[0], EV_PAD, 0)
    vdst = _pad_idx(voxel_edge_index[1], EV_PAD, N_V)
    ce0 = _pad_idx(cross_edge_index[0], EC_PAD, 0)
    ce1g = _pad_idx(cross_edge_index[1], EC_PAD, 0)
    ce1s = _pad_idx(cross_edge_index[1], EC_PAD, N_V)
    cl3 = node_cluster.reshape(NBLK, 1, BLK)
    lvl3 = voxel_level.reshape(NBLK, 1, BLK)

    gumbel = {}
    for li in (1, 3):
        u = jax.random.uniform(jax.random.fold_in(jax.random.key(42), li),
                               (E_C,), minval=1e-9, maxval=1.0, dtype=f32)
        gumbel[li] = (-jnp.log(-jnp.log(u))).reshape(E_C // H, H)

    deg = _sc_count(EP_PAD, pdst)
    d0 = deg[0][:N_P]
    d1 = deg[1][:N_P]

    pw = params["p_enc"]["W"]
    x = _enc2(local_x, program_noise, pw[:128], pw[128:], params["p_enc"]["b"])
    for l in range(P_STEPS):
        wm = params["p_msg"][l]["W"]
        a, b = _pre2(x, wm[:H], wm[H:], params["p_msg"][l]["b"])
        ms = _sc_edge_msg(EP_PAD, _pad_rows(a), _pad_rows(b), pdst, psrc)
        cs, cn = _csum(cl3, x)
        wu = params["p_upd"][l]["W"]
        cm3 = _cmw3(cs, cn, wu[2 * H:])
        x = _pupd(x, ms[0][:N_P], ms[1][:N_P], d0, d1, cl3, cm3, node_ratio,
                  wu[:H], wu[H:2 * H], params["p_upd"][l]["b"])

    pe = _pe_kernel()
    vw = params["v_enc"]["W"]
    v = _venc(voxel_x, voxel_noise, lvl3, pe, vw[:128], vw[128:],
              params["v_enc"]["b"])
    xpad = _pad_rows(x)
    p = params["ptr"]
    for li in range(V_STEPS):
        wm = params["v_msg"][li]["W"]
        a, b = _vpre(v, lvl3, pe, wm[:H], wm[H:2 * H], wm[2 * H:],
                     params["v_msg"][li]["b"])
        ms = _sc_edge_msg(EV_PAD, _pad_rows(a), _pad_rows(b), vdst, vsrc)
        wu = params["v_upd"][li]["W"]
        v = _vupd(v, ms[0][:N_V], ms[1][:N_V], wu[:H], wu[H:],
                  params["v_upd"][li]["b"])
        if (li + 1) % 2 == 0:
            mask, pp, qq = _ptrprep(
                x, v, p["m1"]["W"], p["m1"]["b"], p["m2"]["W"], p["m2"]["b"],
                p["Wp"]["W"], p["Wp"]["b"] + p["Wv"]["b"], p["Wv"]["W"])
            e_edge = _sc_ptr_e(EC_PAD, pp, qq, ce0, ce1g, p["theta"][:, 0])
            e2d = e_edge[:E_C].reshape(E_C // H, H)
            w2d, ssum = _softmax_w(e2d, gumbel[li])
            wpad = jnp.concatenate(
                [w2d.reshape(E_C),
                 jnp.zeros((EC_PAD - E_C + 2 * CHUNK,), f32)])
            sc = _sc_ptr_scatter(EC_PAD, xpad, wpad, ce0, ce1s)
            v = _ptr_final(v, mask, sc[0][:N_V], sc[1][:N_V], ssum)
    return v
